# Initial kernel scaffold; baseline (speedup 1.0000x reference)
import jax
import jax.numpy as jnp
from jax.experimental import pallas as pl


def kernel(x, edge_index, edge_type, basis1, comb1, q1, k1, b1, basis2, comb2, q2, k2, b2, basis3, comb3, q3, k3, b3):
    raise NotImplementedError("write your pallas kernel here")


# speedup vs baseline: 16.7747x; 16.7747x over previous
import functools
import jax
import jax.numpy as jnp
from jax import lax
from jax.experimental import pallas as pl
from jax.experimental.pallas import tpu as pltpu
from jax.experimental.pallas import tpu_sc as plsc

N_NODES = 10000
R = 8
NB = 4

NC = 2
NS = 16
NW = NC * NS
CHUNK = 64
CH_PER_W = 158
EW = CH_PER_W * CHUNK
EPAD = NW * EW
NACC = 10240
NEG = -1e30


def _dense_body(dout, h, x_ref, basis_ref, comb_ref, q_ref, k_ref,
                xw_ref, lg_ref):
    x = x_ref[...]
    bn = x.shape[0]
    pad = jnp.full((bn, 16 - h), NEG, jnp.float32)
    zer = jnp.full((bn, 96), NEG, jnp.float32)
    for r in range(R):
        w = comb_ref[r, 0] * basis_ref[0]
        for b in range(1, NB):
            w = w + comb_ref[r, b] * basis_ref[b]
        xw = jnp.dot(x, w, preferred_element_type=jnp.float32)
        if dout < 128:
            xw_ref[:, r * 128:(r + 1) * 128] = jnp.concatenate(
                [xw, jnp.zeros((bn, 128 - dout), jnp.float32)], axis=1)
        else:
            xw_ref[:, r * dout:(r + 1) * dout] = xw
        asr = jnp.dot(xw, k_ref[...], preferred_element_type=jnp.float32)
        ads = jnp.dot(xw, q_ref[...], preferred_element_type=jnp.float32)
        lg_ref[:, r * 128:(r + 1) * 128] = jnp.concatenate(
            [asr, pad, ads, pad, zer], axis=1)


def _dense_tables(x, basis, comb, q, k, din, dout, h):
    bn = 1000
    out = pl.pallas_call(
        functools.partial(_dense_body, dout, h),
        grid=(N_NODES // bn,),
        in_specs=[
            pl.BlockSpec((bn, din), lambda i: (i, 0)),
            pl.BlockSpec((NB, din, dout), lambda i: (0, 0, 0)),
            pl.BlockSpec(memory_space=pltpu.SMEM),
            pl.BlockSpec((dout, h), lambda i: (0, 0)),
            pl.BlockSpec((dout, h), lambda i: (0, 0)),
        ],
        out_specs=[
            pl.BlockSpec((bn, R * 128), lambda i: (i, 0)),
            pl.BlockSpec((bn, R * 128), lambda i: (i, 0)),
        ],
        out_shape=[
            jax.ShapeDtypeStruct((N_NODES, R * 128), jnp.float32),
            jax.ShapeDtypeStruct((N_NODES, R * 128), jnp.float32),
        ],
    )(x, basis, comb, q, k)
    return (out[0].reshape(N_NODES * R, 128),
            out[1].reshape(N_NODES * R, 128))


def _finalize_body(dout, heads, relu, num_ref, den_ref, bias_ref, out_ref):
    num = (num_ref[0] + num_ref[1])[:, :dout]
    den = den_ref[0] + den_ref[1] + 1e-16
    cd = dout // heads
    bn = num.shape[0]
    scale = jnp.concatenate(
        [jnp.broadcast_to(den[:, hh:hh + 1], (bn, cd)) for hh in range(heads)],
        axis=1)
    v = num / scale + bias_ref[...][None, :]
    if relu:
        v = jnp.maximum(v, 0.0)
    out_ref[...] = v


def _finalize(num_part, den_part, bias, dout, heads, relu):
    bn = 2000
    return pl.pallas_call(
        functools.partial(_finalize_body, dout, heads, relu),
        grid=(N_NODES // bn,),
        in_specs=[
            pl.BlockSpec((2, bn, 128), lambda i: (0, i, 0)),
            pl.BlockSpec((2, bn, 16), lambda i: (0, i, 0)),
            pl.BlockSpec((dout,), lambda i: (0,)),
        ],
        out_specs=pl.BlockSpec((bn, dout), lambda i: (i, 0)),
        out_shape=jax.ShapeDtypeStruct((N_NODES, dout), jnp.float32),
    )(num_part, den_part, bias)


_SPLAT_DNUMS = lax.GatherDimensionNumbers(
    offset_dims=(), collapsed_slice_dims=(0,), start_index_map=(0,))


def _splat(vec, lane):
    idx = jnp.full((16, 1), lane, jnp.int32)
    return lax.gather(vec, idx, dimension_numbers=_SPLAT_DNUMS,
                      slice_sizes=(1,),
                      mode=lax.GatherScatterMode.PROMISE_IN_BOUNDS)


def _sc_body(dout, heads, src_hbm, et_hbm, dst_hbm, lg_hbm, xw_hbm,
             num_out, den_out,
             src_v, et_v, dst_v, idx_v, jdx_v, row_v,
             ai_rows, aj_rows, xw_rows, ex_rows, num_sh, den_sh, sem):
    cid = lax.axis_index("c")
    sid = lax.axis_index("s")
    wid = cid * NS + sid
    rows_per_sub = NACC // NS
    gph = (dout // 16) // heads
    lane = lax.iota(jnp.int32, 16)

    def zb(i, _):
        for g in range(128 // 16):
            xw_rows[i, pl.ds(g * 16, 16)] = jnp.zeros((16,), jnp.float32)
        ex_rows[i, :] = jnp.zeros((16,), jnp.float32)
        return 0
    lax.fori_loop(0, CHUNK, zb, 0)

    def zcp(j, _):
        row = sid * rows_per_sub + j * CHUNK

        def rix(kk, _):
            row_v[pl.ds(kk * 16, 16)] = row + kk * 16 + lane
            return 0
        lax.fori_loop(0, CHUNK // 16, rix, 0)
        pltpu.sync_copy(xw_rows, num_sh.at[row_v])
        pltpu.sync_copy(ex_rows, den_sh.at[row_v])
        return 0
    lax.fori_loop(0, rows_per_sub // CHUNK, zcp, 0)
    plsc.subcore_barrier()

    def chunk(c, _):
        base = wid * EW + c * CHUNK
        pltpu.sync_copy(src_hbm.at[pl.ds(base, CHUNK)], src_v)
        pltpu.sync_copy(et_hbm.at[pl.ds(base, CHUNK)], et_v)
        pltpu.sync_copy(dst_hbm.at[pl.ds(base, CHUNK)], dst_v)

        def idxb(kk, _):
            s = src_v[pl.ds(kk * 16, 16)]
            t = et_v[pl.ds(kk * 16, 16)]
            d = dst_v[pl.ds(kk * 16, 16)]
            idx_v[pl.ds(kk * 16, 16)] = s * R + t
            jdx_v[pl.ds(kk * 16, 16)] = jnp.minimum(d, N_NODES - 1) * R + t
            return 0
        lax.fori_loop(0, CHUNK // 16, idxb, 0)

        cps = [
            pltpu.async_copy(lg_hbm.at[idx_v], ai_rows, sem),
            pltpu.async_copy(lg_hbm.at[jdx_v], aj_rows, sem),
            pltpu.async_copy(xw_hbm.at[idx_v], xw_rows, sem),
        ]
        for cp in cps:
            cp.wait()

        def edge(i, _):
            a = ai_rows[i, pl.ds(0, 16)] + aj_rows[i, pl.ds(16, 16)]
            e16 = jnp.exp(jnp.where(a >= 0, a, 0.2 * a))
            ex_rows[i, :] = e16
            for hh in range(heads):
                sc = _splat(e16, hh)
                for g in range(hh * gph, (hh + 1) * gph):
                    xw_rows[i, pl.ds(g * 16, 16)] = (
                        xw_rows[i, pl.ds(g * 16, 16)] * sc)
            return 0
        lax.fori_loop(0, CHUNK, edge, 0)

        pltpu.sync_copy(ex_rows, den_sh.at[dst_v], add=True)
        pltpu.sync_copy(xw_rows, num_sh.at[dst_v], add=True)
        return 0
    lax.fori_loop(0, CH_PER_W, chunk, 0)
    plsc.subcore_barrier()

    def ocp(j, _):
        row = sid * rows_per_sub + j * CHUNK

        def rix(kk, _):
            row_v[pl.ds(kk * 16, 16)] = row + kk * 16 + lane
            return 0
        lax.fori_loop(0, CHUNK // 16, rix, 0)
        pltpu.sync_copy(num_sh.at[row_v], xw_rows)
        pltpu.sync_copy(xw_rows, num_out.at[cid, pl.ds(row, CHUNK)])
        pltpu.sync_copy(den_sh.at[row_v], ex_rows)
        pltpu.sync_copy(ex_rows, den_out.at[cid, pl.ds(row, CHUNK)])
        return 0
    lax.fori_loop(0, rows_per_sub // CHUNK, ocp, 0)


def _sc_pass(src_p, et_p, dst_p, lg, xw, dout, heads):
    mesh = plsc.VectorSubcoreMesh(core_axis_name="c", subcore_axis_name="s")
    fn = pl.kernel(
        functools.partial(_sc_body, dout, heads),
        out_type=[
            jax.ShapeDtypeStruct((NC, NACC, 128), jnp.float32),
            jax.ShapeDtypeStruct((NC, NACC, 16), jnp.float32),
        ],
        mesh=mesh,
        scratch_types=[
            pltpu.VMEM((CHUNK,), jnp.int32),
            pltpu.VMEM((CHUNK,), jnp.int32),
            pltpu.VMEM((CHUNK,), jnp.int32),
            pltpu.VMEM((CHUNK,), jnp.int32),
            pltpu.VMEM((CHUNK,), jnp.int32),
            pltpu.VMEM((CHUNK,), jnp.int32),
            pltpu.VMEM((CHUNK, 128), jnp.float32),
            pltpu.VMEM((CHUNK, 128), jnp.float32),
            pltpu.VMEM((CHUNK, 128), jnp.float32),
            pltpu.VMEM((CHUNK, 16), jnp.float32),
            pltpu.VMEM_SHARED((NACC, 128), jnp.float32),
            pltpu.VMEM_SHARED((NACC, 16), jnp.float32),
            pltpu.SemaphoreType.DMA,
        ],
    )
    return fn(src_p, et_p, dst_p, lg, xw)


def _layer(x, src_p, et_p, dst_p, basis, comb, q, k, bias, din, dout, heads,
           relu):
    xw, lg = _dense_tables(x, basis, comb, q, k, din, dout, heads)
    num_part, den_part = _sc_pass(src_p, et_p, dst_p, lg, xw, dout, heads)
    return _finalize(num_part, den_part, bias, dout, heads, relu)


def kernel(x, edge_index, edge_type, basis1, comb1, q1, k1, b1,
           basis2, comb2, q2, k2, b2, basis3, comb3, q3, k3, b3):
    src = edge_index[0]
    dst = edge_index[1]
    npad = EPAD - src.shape[0]
    src_p = jnp.concatenate([src, jnp.zeros((npad,), jnp.int32)])
    et_p = jnp.concatenate([edge_type, jnp.zeros((npad,), jnp.int32)])
    dst_p = jnp.concatenate([dst, jnp.full((npad,), N_NODES, jnp.int32)])

    h = _layer(x, src_p, et_p, dst_p, basis1, comb1, q1, k1, b1,
               128, 128, 4, True)
    h = _layer(h, src_p, et_p, dst_p, basis2, comb2, q2, k2, b2,
               128, 128, 1, True)
    out = _layer(h, src_p, et_p, dst_p, basis3, comb3, q3, k3, b3,
                 128, 64, 1, False)
    return out


# speedup vs baseline: 21.8764x; 1.3041x over previous
import functools
import jax
import jax.numpy as jnp
from jax import lax
from jax.experimental import pallas as pl
from jax.experimental.pallas import tpu as pltpu
from jax.experimental.pallas import tpu_sc as plsc

N_NODES = 10000
R = 8
NB = 4

NC = 2
NS = 16
NW = NC * NS
CHUNK = 40
CH_PER_W = 254
EW = CH_PER_W * CHUNK
EPAD = NW * EW
NACC = 10240
NEG = -1e30


def _dense_body(dout, h, x_ref, basis_ref, comb_ref, q_ref, k_ref,
                xw_ref, lg_ref):
    x = x_ref[...]
    bn = x.shape[0]
    pad = jnp.full((bn, 16 - h), NEG, jnp.float32)
    zer = jnp.full((bn, 96), NEG, jnp.float32)
    for r in range(R):
        w = comb_ref[r, 0] * basis_ref[0]
        for b in range(1, NB):
            w = w + comb_ref[r, b] * basis_ref[b]
        xw = jnp.dot(x, w, preferred_element_type=jnp.float32)
        if dout < 128:
            xw_ref[:, r * 128:(r + 1) * 128] = jnp.concatenate(
                [xw, jnp.zeros((bn, 128 - dout), jnp.float32)], axis=1)
        else:
            xw_ref[:, r * dout:(r + 1) * dout] = xw
        asr = jnp.dot(xw, k_ref[...], preferred_element_type=jnp.float32)
        ads = jnp.dot(xw, q_ref[...], preferred_element_type=jnp.float32)
        lg_ref[:, r * 128:(r + 1) * 128] = jnp.concatenate(
            [asr, pad, ads, pad, zer], axis=1)


def _dense_tables(x, basis, comb, q, k, din, dout, h):
    bn = 1000
    out = pl.pallas_call(
        functools.partial(_dense_body, dout, h),
        grid=(N_NODES // bn,),
        in_specs=[
            pl.BlockSpec((bn, din), lambda i: (i, 0)),
            pl.BlockSpec((NB, din, dout), lambda i: (0, 0, 0)),
            pl.BlockSpec(memory_space=pltpu.SMEM),
            pl.BlockSpec((dout, h), lambda i: (0, 0)),
            pl.BlockSpec((dout, h), lambda i: (0, 0)),
        ],
        out_specs=[
            pl.BlockSpec((bn, R * 128), lambda i: (i, 0)),
            pl.BlockSpec((bn, R * 128), lambda i: (i, 0)),
        ],
        out_shape=[
            jax.ShapeDtypeStruct((N_NODES, R * 128), jnp.float32),
            jax.ShapeDtypeStruct((N_NODES, R * 128), jnp.float32),
        ],
    )(x, basis, comb, q, k)
    return (out[0].reshape(N_NODES * R, 128),
            out[1].reshape(N_NODES * R, 128))


def _finalize_body(dout, heads, relu, num_ref, den_ref, bias_ref, out_ref):
    num = (num_ref[0] + num_ref[1])[:, :dout]
    den = den_ref[0] + den_ref[1] + 1e-16
    cd = dout // heads
    bn = num.shape[0]
    scale = jnp.concatenate(
        [jnp.broadcast_to(den[:, hh:hh + 1], (bn, cd)) for hh in range(heads)],
        axis=1)
    v = num / scale + bias_ref[...][None, :]
    if relu:
        v = jnp.maximum(v, 0.0)
    out_ref[...] = v


def _finalize(num_part, den_part, bias, dout, heads, relu):
    bn = 2000
    return pl.pallas_call(
        functools.partial(_finalize_body, dout, heads, relu),
        grid=(N_NODES // bn,),
        in_specs=[
            pl.BlockSpec((2, bn, 128), lambda i: (0, i, 0)),
            pl.BlockSpec((2, bn, 16), lambda i: (0, i, 0)),
            pl.BlockSpec((dout,), lambda i: (0,)),
        ],
        out_specs=pl.BlockSpec((bn, dout), lambda i: (i, 0)),
        out_shape=jax.ShapeDtypeStruct((N_NODES, dout), jnp.float32),
    )(num_part, den_part, bias)


_SPLAT_DNUMS = lax.GatherDimensionNumbers(
    offset_dims=(), collapsed_slice_dims=(0,), start_index_map=(0,))


def _splat(vec, lane):
    idx = jnp.full((16, 1), lane, jnp.int32)
    return lax.gather(vec, idx, dimension_numbers=_SPLAT_DNUMS,
                      slice_sizes=(1,),
                      mode=lax.GatherScatterMode.PROMISE_IN_BOUNDS)


def _sc_body(dout, heads, src_hbm, et_hbm, dst_hbm, lg_hbm, xw_hbm,
             num_out, den_out,
             src_v0, et_v0, dst_v0, idx_v0, jdx_v0,
             src_v1, et_v1, dst_v1, idx_v1, jdx_v1, row_v,
             ai0, aj0, xw0, ai1, aj1, xw1, ex_rows,
             num_sh, den_sh, sem0, sem1):
    cid = lax.axis_index("c")
    sid = lax.axis_index("s")
    wid = cid * NS + sid
    rows_per_sub = NACC // NS
    gph = (dout // 16) // heads
    lane = lax.iota(jnp.int32, 16)
    bufs = ((src_v0, et_v0, dst_v0, idx_v0, jdx_v0, ai0, aj0, xw0, sem0),
            (src_v1, et_v1, dst_v1, idx_v1, jdx_v1, ai1, aj1, xw1, sem1))

    def zb(i, _):
        for g in range(128 // 16):
            xw0[i, pl.ds(g * 16, 16)] = jnp.zeros((16,), jnp.float32)
        ex_rows[i, :] = jnp.zeros((16,), jnp.float32)
        return 0
    lax.fori_loop(0, CHUNK, zb, 0)

    def zcp(j, _):
        row = sid * rows_per_sub + j * CHUNK

        def rix(kk, _):
            row_v[pl.ds(kk * 16, 16)] = row + kk * 16 + lane
            return 0
        lax.fori_loop(0, (CHUNK + 15) // 16, rix, 0)
        pltpu.sync_copy(xw0, num_sh.at[row_v])
        pltpu.sync_copy(ex_rows, den_sh.at[row_v])
        return 0
    lax.fori_loop(0, rows_per_sub // CHUNK, zcp, 0)
    plsc.subcore_barrier()

    def issue(c, b):
        (s_v, e_v, d_v, i_v, j_v, ai, aj, xw, sem) = bufs[b]
        base = wid * EW + c * CHUNK
        pltpu.sync_copy(src_hbm.at[pl.ds(base, CHUNK)], s_v)
        pltpu.sync_copy(et_hbm.at[pl.ds(base, CHUNK)], e_v)
        pltpu.sync_copy(dst_hbm.at[pl.ds(base, CHUNK)], d_v)

        def idxb(kk, _):
            s = s_v[pl.ds(kk * 16, 16)]
            t = e_v[pl.ds(kk * 16, 16)]
            d = d_v[pl.ds(kk * 16, 16)]
            i_v[pl.ds(kk * 16, 16)] = s * R + t
            j_v[pl.ds(kk * 16, 16)] = jnp.minimum(d, N_NODES - 1) * R + t
            return 0
        lax.fori_loop(0, (CHUNK + 15) // 16, idxb, 0)
        pltpu.async_copy(lg_hbm.at[i_v], ai, sem)
        pltpu.async_copy(lg_hbm.at[j_v], aj, sem)
        pltpu.async_copy(xw_hbm.at[i_v], xw, sem)

    def process(c, b):
        (s_v, e_v, d_v, i_v, j_v, ai, aj, xw, sem) = bufs[b]
        pltpu.make_async_copy(lg_hbm.at[i_v], ai, sem).wait()
        pltpu.make_async_copy(lg_hbm.at[j_v], aj, sem).wait()
        pltpu.make_async_copy(xw_hbm.at[i_v], xw, sem).wait()

        def edge(i, _):
            a = ai[i, pl.ds(0, 16)] + aj[i, pl.ds(16, 16)]
            e16 = jnp.exp(jnp.where(a >= 0, a, 0.2 * a))
            ex_rows[i, :] = e16
            for hh in range(heads):
                sc = _splat(e16, hh)
                for g in range(hh * gph, (hh + 1) * gph):
                    xw[i, pl.ds(g * 16, 16)] = xw[i, pl.ds(g * 16, 16)] * sc
            return 0
        lax.fori_loop(0, CHUNK, edge, 0)
        pltpu.sync_copy(ex_rows, den_sh.at[d_v], add=True)
        pltpu.sync_copy(xw, num_sh.at[d_v], add=True)

    issue(0, 0)
    issue(1, 1)

    def pipe(k, _):
        c0 = 2 * k
        process(c0, 0)

        @pl.when(c0 + 2 < CH_PER_W)
        def _():
            issue(c0 + 2, 0)
        process(c0 + 1, 1)

        @pl.when(c0 + 3 < CH_PER_W)
        def _():
            issue(c0 + 3, 1)
        return 0
    lax.fori_loop(0, CH_PER_W // 2, pipe, 0)
    plsc.subcore_barrier()

    def ocp(j, _):
        row = sid * rows_per_sub + j * CHUNK

        def rix(kk, _):
            row_v[pl.ds(kk * 16, 16)] = row + kk * 16 + lane
            return 0
        lax.fori_loop(0, (CHUNK + 15) // 16, rix, 0)
        pltpu.sync_copy(num_sh.at[row_v], xw0)
        pltpu.sync_copy(xw0, num_out.at[cid, pl.ds(row, CHUNK)])
        pltpu.sync_copy(den_sh.at[row_v], ex_rows)
        pltpu.sync_copy(ex_rows, den_out.at[cid, pl.ds(row, CHUNK)])
        return 0
    lax.fori_loop(0, rows_per_sub // CHUNK, ocp, 0)


def _sc_pass(src_p, et_p, dst_p, lg, xw, dout, heads):
    mesh = plsc.VectorSubcoreMesh(core_axis_name="c", subcore_axis_name="s")
    fn = pl.kernel(
        functools.partial(_sc_body, dout, heads),
        out_type=[
            jax.ShapeDtypeStruct((NC, NACC, 128), jnp.float32),
            jax.ShapeDtypeStruct((NC, NACC, 16), jnp.float32),
        ],
        mesh=mesh,
        scratch_types=(
            [pltpu.VMEM((CHUNK,), jnp.int32)] * 11
            + [
                pltpu.VMEM((CHUNK, 128), jnp.float32),
                pltpu.VMEM((CHUNK, 128), jnp.float32),
                pltpu.VMEM((CHUNK, 128), jnp.float32),
                pltpu.VMEM((CHUNK, 128), jnp.float32),
                pltpu.VMEM((CHUNK, 128), jnp.float32),
                pltpu.VMEM((CHUNK, 128), jnp.float32),
                pltpu.VMEM((CHUNK, 16), jnp.float32),
                pltpu.VMEM_SHARED((NACC, 128), jnp.float32),
                pltpu.VMEM_SHARED((NACC, 16), jnp.float32),
                pltpu.SemaphoreType.DMA,
                pltpu.SemaphoreType.DMA,
            ]
        ),
    )
    return fn(src_p, et_p, dst_p, lg, xw)


def _layer(x, src_p, et_p, dst_p, basis, comb, q, k, bias, din, dout, heads,
           relu):
    xw, lg = _dense_tables(x, basis, comb, q, k, din, dout, heads)
    num_part, den_part = _sc_pass(src_p, et_p, dst_p, lg, xw, dout, heads)
    return _finalize(num_part, den_part, bias, dout, heads, relu)


def kernel(x, edge_index, edge_type, basis1, comb1, q1, k1, b1,
           basis2, comb2, q2, k2, b2, basis3, comb3, q3, k3, b3):
    src = edge_index[0]
    dst = edge_index[1]
    npad = EPAD - src.shape[0]
    src_p = jnp.concatenate([src, jnp.zeros((npad,), jnp.int32)])
    et_p = jnp.concatenate([edge_type, jnp.zeros((npad,), jnp.int32)])
    dst_p = jnp.concatenate([dst, jnp.full((npad,), N_NODES, jnp.int32)])

    h = _layer(x, src_p, et_p, dst_p, basis1, comb1, q1, k1, b1,
               128, 128, 4, True)
    h = _layer(h, src_p, et_p, dst_p, basis2, comb2, q2, k2, b2,
               128, 128, 1, True)
    out = _layer(h, src_p, et_p, dst_p, basis3, comb3, q3, k3, b3,
                 128, 64, 1, False)
    return out
